# baseline (speedup 1.0000x reference)
import functools

import jax
import jax.numpy as jnp
import numpy as np
from jax import lax
from jax.experimental import pallas as pl
from jax.experimental.pallas import tpu as pltpu
from jax.experimental.pallas import tpu_sc as plsc

W_IMG = 512.0
N_HEAD = 6
N_LEVELS = 3
N_POINTS = 4
D_MODEL = 192
D_HEAD = D_MODEL // N_HEAD
STRIDES = (8.0, 16.0, 32.0)
SIGMAS = (4.0, 2.0, 1.0)
LVL_W = (64, 32, 16)
LVL_OFF = (0, 4096, 5120)
NPOS = 5376
J = N_HEAD * N_LEVELS * N_POINTS

TT = 2048
T_TOTAL = 65536
NW = 32
TPW = T_TOTAL // NW
CH = 32


def _np_consts():
    j = np.arange(J)
    l = (j % (N_LEVELS * N_POINTS)) // N_POINTS
    h = j // (N_LEVELS * N_POINTS)
    sig = np.array(SIGMAS, np.float32)[l]
    invstr = (1.0 / np.array(STRIDES, np.float32))[l]
    wl = np.array(LVL_W, np.float32)[l]
    off = np.array(LVL_OFF, np.float32)[l]
    return (sig[None], invstr[None], wl[None], off[None],
            h[None].astype(np.int32), wl[None].astype(np.int32) if False else wl[None])


def _meta_kernel(hx_ref, idx_ref, qc_ref, g_ref,
                 wu_ref, bu_ref, lng_ref, lnb_ref,
                 wdx_ref, wdy_ref, bdx_ref, bdy_ref, wa_ref, ba_ref, sblk_ref,
                 sig_ref, invstr_ref, wl_ref, loff_ref, head_ref,
                 fx_ref, fy_ref, sh_ref, pt_ref,
                 i0_ref, i1_ref, i2_ref, i3_ref,
                 w0_ref, w1_ref, w2_ref, w3_ref):
    b = pl.program_id(0)
    idx = idx_ref[0]
    axi = jnp.remainder(idx, 16).astype(jnp.float32)
    ayi = (idx // 16).astype(jnp.float32)
    apx = axi * 32.0 + 16.0
    apy = ayi * 32.0 + 16.0
    qc = qc_ref[0]
    dxn = (apx - qc[:, 0:1]) / W_IMG
    dyn = (apy - qc[:, 1:2]) / W_IMG
    z = dxn * fx_ref[...] + dyn * fy_ref[...]
    phi = jnp.where(sh_ref[...] > 0.5, jnp.cos(z), jnp.sin(z))

    col = lax.broadcasted_iota(jnp.int32, (TT, 256), 1)
    oh = (col == idx).astype(jnp.float32)
    g_r = jnp.dot(oh, g_ref[0], preferred_element_type=jnp.float32,
                  precision=jax.lax.Precision.HIGHEST)
    xcat = jnp.concatenate([hx_ref[0], g_r, phi], axis=1)
    pre = jnp.dot(xcat, wu_ref[...], preferred_element_type=jnp.float32) + bu_ref[...]
    u = 0.5 * pre * (1.0 + lax.erf(pre * np.float32(0.7071067811865476)))
    m = jnp.mean(u, axis=-1, keepdims=True)
    v = jnp.mean((u - m) * (u - m), axis=-1, keepdims=True)
    u_r = (u - m) / jnp.sqrt(v + 1e-5) * lng_ref[...] + lnb_ref[...]

    logits = jnp.dot(u_r, wa_ref[...], preferred_element_type=jnp.float32) + ba_ref[...]
    e = jnp.exp(logits - jnp.max(logits, axis=-1, keepdims=True))
    attn = e / jnp.dot(e, sblk_ref[...], preferred_element_type=jnp.float32, precision=jax.lax.Precision.HIGHEST)

    offx = jnp.tanh(jnp.dot(u_r, wdx_ref[...], preferred_element_type=jnp.float32)
                    + bdx_ref[...]) * sig_ref[...]
    offy = jnp.tanh(jnp.dot(u_r, wdy_ref[...], preferred_element_type=jnp.float32)
                    + bdy_ref[...]) * sig_ref[...]
    x = apx * invstr_ref[...] + offx
    y = apy * invstr_ref[...] + offy
    wl = wl_ref[...]
    x0 = jnp.floor(x)
    y0 = jnp.floor(y)
    fx = x - x0
    fy = y - y0
    vx0 = ((x0 >= 0.0) & (x0 <= wl - 1.0)).astype(jnp.float32)
    vx1 = ((x0 + 1.0 >= 0.0) & (x0 + 1.0 <= wl - 1.0)).astype(jnp.float32)
    vy0 = ((y0 >= 0.0) & (y0 <= wl - 1.0)).astype(jnp.float32)
    vy1 = ((y0 + 1.0 >= 0.0) & (y0 + 1.0 <= wl - 1.0)).astype(jnp.float32)
    xc0 = jnp.clip(x0, 0.0, wl - 1.0)
    xc1 = jnp.clip(x0 + 1.0, 0.0, wl - 1.0)
    yc0 = jnp.clip(y0, 0.0, wl - 1.0)
    yc1 = jnp.clip(y0 + 1.0, 0.0, wl - 1.0)
    base = loff_ref[...] + b * np.float32(NPOS)
    hd = head_ref[...]
    i0_ref[0] = ((base + yc0 * wl + xc0) * 6.0 + hd).astype(jnp.int32)
    i1_ref[0] = ((base + yc0 * wl + xc1) * 6.0 + hd).astype(jnp.int32)
    i2_ref[0] = ((base + yc1 * wl + xc0) * 6.0 + hd).astype(jnp.int32)
    i3_ref[0] = ((base + yc1 * wl + xc1) * 6.0 + hd).astype(jnp.int32)
    pt = pt_ref[...]
    scatter96 = lambda w: jnp.dot(w, pt, preferred_element_type=jnp.float32, precision=jax.lax.Precision.HIGHEST)
    w0_ref[0] = scatter96(attn * (1.0 - fx) * (1.0 - fy) * vx0 * vy0)
    w1_ref[0] = scatter96(attn * fx * (1.0 - fy) * vx1 * vy0)
    w2_ref[0] = scatter96(attn * (1.0 - fx) * fy * vx0 * vy1)
    w3_ref[0] = scatter96(attn * fx * fy * vx1 * vy1)


def _run_meta(hx, idxf, qcf, g, wu, bu, lng, lnb,
              wdx, wdy, bdx, bdy, wa, ba, sblk,
              sig, invstr, wl, loff, hd, fxr, fyr, shr, ptm, interpret=False):
    B = g.shape[0]
    n_t = hx.shape[1] // TT
    rep = lambda shp: pl.BlockSpec(shp, lambda b, i: (0, 0))
    out_sp = pl.BlockSpec((1, TT, J), lambda b, i: (b, i, 0))
    out_sp96 = pl.BlockSpec((1, TT, 96), lambda b, i: (b, i, 0))
    out_st = jax.ShapeDtypeStruct((B, n_t * TT, J), jnp.int32)
    out_sf = jax.ShapeDtypeStruct((B, n_t * TT, 96), jnp.float32)
    return pl.pallas_call(
        _meta_kernel,
        grid=(B, n_t),
        in_specs=[
            pl.BlockSpec((1, TT, D_MODEL), lambda b, i: (b, i, 0)),
            pl.BlockSpec((1, TT, 1), lambda b, i: (b, i, 0)),
            pl.BlockSpec((1, TT, 2), lambda b, i: (b, i, 0)),
            pl.BlockSpec((1, 256, D_MODEL), lambda b, i: (b, 0, 0)),
            rep((416, D_MODEL)),
            rep((1, D_MODEL)), rep((1, D_MODEL)), rep((1, D_MODEL)),
            rep((D_MODEL, J)), rep((D_MODEL, J)), rep((1, J)), rep((1, J)),
            rep((D_MODEL, J)), rep((1, J)), rep((J, J)),
            rep((1, J)), rep((1, J)), rep((1, J)), rep((1, J)), rep((1, J)),
            rep((1, 32)), rep((1, 32)), rep((1, 32)), rep((J, 96)),
        ],
        out_specs=[out_sp] * 4 + [out_sp96] * 4,
        out_shape=[out_st] * 4 + [out_sf] * 4,
        interpret=interpret,
    )(hx, idxf, qcf, g, wu, bu, lng, lnb,
      wdx, wdy, bdx, bdy, wa, ba, sblk,
      sig, invstr, wl, loff, hd, fxr, fyr, shr, ptm)


def _sc_gather(table, i0, i1, i2, i3, w0, w1, w2, w3):
    mesh = plsc.VectorSubcoreMesh(core_axis_name="c", subcore_axis_name="s")

    @functools.partial(
        pl.kernel, mesh=mesh,
        compiler_params=pltpu.CompilerParams(use_tc_tiling_on_sc=False, needs_layout_passes=False),
        out_type=jax.ShapeDtypeStruct((T_TOTAL, D_MODEL), jnp.float32),
        scratch_types=(
            [pltpu.VMEM((CH, J), jnp.int32) for _ in range(4)]
            + [pltpu.VMEM((CH, N_HEAD, 16), jnp.float32) for _ in range(4)]
            + [pltpu.VMEM((J, D_HEAD), jnp.bfloat16) for _ in range(16)]
            + [pltpu.VMEM((CH, D_MODEL), jnp.float32),
               pltpu.SemaphoreType.DMA,
               pltpu.SemaphoreType.DMA, pltpu.SemaphoreType.DMA,
               pltpu.SemaphoreType.DMA, pltpu.SemaphoreType.DMA]
        ),
    )
    def k(tab_hbm, i0h, i1h, i2h, i3h, w0h, w1h, w2h, w3h, out_hbm,
          ib0, ib1, ib2, ib3, wb0, wb1, wb2, wb3,
          ra0, ra1, ra2, ra3, rb0, rb1, rb2, rb3,
          rc0, rc1, rc2, rc3, rd0, rd1, rd2, rd3,
          ob, msem, gsa, gsb, gsc, gsd):
        wid = lax.axis_index("s") * 2 + lax.axis_index("c")
        tbase = wid * TPW
        ibufs = (ib0, ib1, ib2, ib3)
        wbufs = (wb0, wb1, wb2, wb3)
        rsets = ((ra0, ra1, ra2, ra3), (rb0, rb1, rb2, rb3),
                 (rc0, rc1, rc2, rc3), (rd0, rd1, rd2, rd3))
        sems = (gsa, gsb, gsc, gsd)

        def fire(i, s):
            for t in range(4):
                pltpu.async_copy(tab_hbm.at[ibufs[t].at[i]], rsets[s][t], sems[s])

        def drain(s):
            for t in range(4):
                pltpu.make_async_copy(tab_hbm.at[ibufs[t].at[0]], rsets[s][t], sems[s]).wait()

        def compute(i, rows):
            for h in range(N_HEAD):
                los = []
                his = []
                for t in range(4):
                    wv = wbufs[t][i, h]
                    lo = jnp.zeros((16,), jnp.float32)
                    hi = jnp.zeros((16,), jnp.float32)
                    for lm in range(12):
                        j = h * 12 + lm
                        w = wv[lm]
                        vi = plsc.bitcast(rows[t][j], jnp.int32)
                        lo = lo + w * plsc.bitcast(vi << 16, jnp.float32)
                        hi = hi + w * plsc.bitcast(vi, jnp.float32)
                    los.append(lo)
                    his.append(hi)
                ob[i, h * D_HEAD:h * D_HEAD + 16] = (los[0] + los[1]) + (los[2] + los[3])
                ob[i, h * D_HEAD + 16:h * D_HEAD + 32] = (his[0] + his[1]) + (his[2] + his[3])

        def chunk_body(c, _):
            base = tbase + c * CH
            srcs = (i0h, i1h, i2h, i3h, w0h, w1h, w2h, w3h)
            dsts = (ib0, ib1, ib2, ib3, wb0, wb1, wb2, wb3)
            for s, d in zip(srcs, dsts):
                pltpu.async_copy(s.at[pl.ds(base, CH)], d, msem)
            for s, d in zip(srcs, dsts):
                pltpu.make_async_copy(s.at[pl.ds(base, CH)], d, msem).wait()
            fire(0, 0)
            fire(1, 1)
            fire(2, 2)

            def quad_body(p, _):
                i0q = 4 * p
                for s in range(4):
                    nxt = i0q + s + 3
                    if s == 0:
                        fire(nxt, 3)
                    else:
                        @pl.when(nxt < CH)
                        def _(nxt=nxt, s=s):
                            fire(nxt, s - 1)
                    drain(s)
                    compute(i0q + s, rsets[s])
                return 0

            lax.fori_loop(0, CH // 4, quad_body, 0)
            pltpu.async_copy(ob, out_hbm.at[pl.ds(base, CH)], msem)
            pltpu.make_async_copy(ob, out_hbm.at[pl.ds(base, CH)], msem).wait()
            return 0

        lax.fori_loop(0, TPW // CH, chunk_body, 0)

    return k(table, i0, i1, i2, i3, w0, w1, w2, w3)


def _proj_kernel(hr_ref, wo_ref, bo_ref, out_ref):
    out_ref[...] = jnp.dot(hr_ref[...], wo_ref[...],
                           preferred_element_type=jnp.float32) + bo_ref[...]


def _run_proj(hr, w_o, bias, interpret=False):
    return pl.pallas_call(
        _proj_kernel,
        grid=(T_TOTAL // 2048,),
        in_specs=[
            pl.BlockSpec((2048, D_MODEL), lambda i: (i, 0)),
            pl.BlockSpec((D_MODEL, D_MODEL), lambda i: (0, 0)),
            pl.BlockSpec((1, D_MODEL), lambda i: (0, 0)),
        ],
        out_specs=pl.BlockSpec((2048, D_MODEL), lambda i: (i, 0)),
        out_shape=jax.ShapeDtypeStruct((T_TOTAL, D_MODEL), jnp.float32),
        interpret=interpret,
    )(hr, w_o, bias)


def _prep(h, top_indices, query_coords, g, L2_proj, L3_proj, L4_proj,
          w_u, b_u, w_delta, b_delta):
    B, K, d = h.shape
    R = top_indices.shape[2]
    hx = jnp.broadcast_to(h[:, :, None, :], (B, K, R, d)).reshape(B, K * R, d)
    idxf = top_indices.reshape(B, K * R, 1)
    qcf = jnp.broadcast_to(query_coords[:, :, None, :], (B, K, R, 2)).reshape(B, K * R, 2)
    tabs = []
    for F in (L2_proj, L3_proj, L4_proj):
        Hf, Wf = F.shape[2], F.shape[3]
        tabs.append(jnp.transpose(F, (0, 2, 3, 1)).reshape(B, Hf * Wf, d))
    table = jnp.concatenate(tabs, axis=1).reshape(B * NPOS * N_HEAD, D_HEAD)
    perm = np.empty(D_HEAD, np.int64)
    perm[0::2] = np.arange(16)
    perm[1::2] = np.arange(16) + 16
    table = table[:, perm].astype(jnp.bfloat16)
    wdx = w_delta[:, 0::2]
    wdy = w_delta[:, 1::2]
    bdx = b_delta[0::2].reshape(1, J)
    bdy = b_delta[1::2].reshape(1, J)
    bu = b_u.reshape(1, d)
    return hx, idxf, qcf, table, bu, wdx, wdy, bdx, bdy


def _consts():
    j = np.arange(J)
    l = (j % (N_LEVELS * N_POINTS)) // N_POINTS
    hh = j // (N_LEVELS * N_POINTS)
    sig = np.array(SIGMAS, np.float32)[l][None]
    invstr = (1.0 / np.array(STRIDES, np.float32))[l][None]
    wl = np.array(LVL_W, np.float32)[l][None]
    loff = np.array(LVL_OFF, np.float32)[l][None]
    hd = hh[None].astype(np.float32)
    sb = np.kron(np.eye(N_HEAD, dtype=np.float32), np.ones((12, 12), np.float32))
    fr = 2.0 ** np.arange(8)
    fxr = 2.0 * np.pi * np.concatenate([fr, fr, np.zeros(16)])[None].astype(np.float32)
    fyr = 2.0 * np.pi * np.concatenate([np.zeros(16), fr, fr])[None].astype(np.float32)
    shr = np.concatenate([np.zeros(8), np.ones(8),
                          np.zeros(8), np.ones(8)])[None].astype(np.float32)
    pt = np.zeros((J, 96), np.float32)
    for jj in range(J):
        hq, lm = jj // 12, jj % 12
        pt[jj, hq * 16 + lm] = 1.0
    return (jnp.asarray(sig), jnp.asarray(invstr), jnp.asarray(wl),
            jnp.asarray(loff), jnp.asarray(hd), jnp.asarray(sb),
            jnp.asarray(fxr), jnp.asarray(fyr), jnp.asarray(shr),
            jnp.asarray(pt))


def kernel(h, top_indices, query_coords, g, L2_proj, L3_proj, L4_proj,
           w_u, b_u, ln_g, ln_b, w_delta, b_delta, w_a, b_a, w_o, b_o, e_deform):
    B, K, d = h.shape
    R = top_indices.shape[2]
    (hx, idxf, qcf, table, bu,
     wdx, wdy, bdx, bdy) = _prep(h, top_indices, query_coords, g,
                                 L2_proj, L3_proj, L4_proj, w_u, b_u,
                                 w_delta, b_delta)
    sig, invstr, wl, loff, hd, sb, fxr, fyr, shr, ptm = _consts()
    i0, i1, i2, i3, w0, w1, w2, w3 = _run_meta(
        hx, idxf, qcf, g, w_u, bu,
        ln_g.reshape(1, d), ln_b.reshape(1, d),
        wdx, wdy, bdx, bdy, w_a, b_a.reshape(1, J), sb,
        sig, invstr, wl, loff, hd, fxr, fyr, shr, ptm)
    rs = lambda a: a.reshape(T_TOTAL, J)
    rw = lambda a: a.reshape(T_TOTAL, N_HEAD, 16)
    hr = _sc_gather(table, rs(i0), rs(i1), rs(i2), rs(i3),
                    rw(w0), rw(w1), rw(w2), rw(w3))
    bias = (b_o + e_deform.reshape(d)).reshape(1, d)
    out = _run_proj(hr, w_o, bias)
    return out.reshape(B, K, R, d)


# speedup vs baseline: 1.3102x; 1.3102x over previous
import functools

import jax
import jax.numpy as jnp
import numpy as np
from jax import lax
from jax.experimental import pallas as pl
from jax.experimental.pallas import tpu as pltpu
from jax.experimental.pallas import tpu_sc as plsc

W_IMG = 512.0
N_HEAD = 6
N_LEVELS = 3
N_POINTS = 4
D_MODEL = 192
D_HEAD = D_MODEL // N_HEAD
STRIDES = (8.0, 16.0, 32.0)
SIGMAS = (4.0, 2.0, 1.0)
LVL_W = (64, 32, 16)
LVL_OFF = (0, 4096, 5120)
NPOS = 5376
J = N_HEAD * N_LEVELS * N_POINTS

TT = 2048
T_TOTAL = 65536
NW = 32
TPW = T_TOTAL // NW
CH = 16


def _np_consts():
    j = np.arange(J)
    l = (j % (N_LEVELS * N_POINTS)) // N_POINTS
    h = j // (N_LEVELS * N_POINTS)
    sig = np.array(SIGMAS, np.float32)[l]
    invstr = (1.0 / np.array(STRIDES, np.float32))[l]
    wl = np.array(LVL_W, np.float32)[l]
    off = np.array(LVL_OFF, np.float32)[l]
    return (sig[None], invstr[None], wl[None], off[None],
            h[None].astype(np.int32), wl[None].astype(np.int32) if False else wl[None])


def _meta_kernel(hx_ref, idx_ref, qc_ref, g_ref,
                 wu_ref, bu_ref, lng_ref, lnb_ref,
                 wdx_ref, wdy_ref, bdx_ref, bdy_ref, wa_ref, ba_ref, sblk_ref,
                 sig_ref, invstr_ref, wl_ref, loff_ref, head_ref,
                 fx_ref, fy_ref, sh_ref, pt_ref,
                 i0_ref, i1_ref, i2_ref, i3_ref,
                 w0_ref, w1_ref, w2_ref, w3_ref):
    b = pl.program_id(0)
    idx = idx_ref[0]
    axi = jnp.remainder(idx, 16).astype(jnp.float32)
    ayi = (idx // 16).astype(jnp.float32)
    apx = axi * 32.0 + 16.0
    apy = ayi * 32.0 + 16.0
    qc = qc_ref[0]
    dxn = (apx - qc[:, 0:1]) / W_IMG
    dyn = (apy - qc[:, 1:2]) / W_IMG
    z = dxn * fx_ref[...] + dyn * fy_ref[...]
    phi = jnp.where(sh_ref[...] > 0.5, jnp.cos(z), jnp.sin(z))

    col = lax.broadcasted_iota(jnp.int32, (TT, 256), 1)
    oh = (col == idx).astype(jnp.float32)
    g_r = jnp.dot(oh, g_ref[0], preferred_element_type=jnp.float32,
                  precision=jax.lax.Precision.HIGHEST)
    xcat = jnp.concatenate([hx_ref[0], g_r, phi], axis=1)
    pre = jnp.dot(xcat, wu_ref[...], preferred_element_type=jnp.float32) + bu_ref[...]
    u = 0.5 * pre * (1.0 + lax.erf(pre * np.float32(0.7071067811865476)))
    m = jnp.mean(u, axis=-1, keepdims=True)
    v = jnp.mean((u - m) * (u - m), axis=-1, keepdims=True)
    u_r = (u - m) / jnp.sqrt(v + 1e-5) * lng_ref[...] + lnb_ref[...]

    logits = jnp.dot(u_r, wa_ref[...], preferred_element_type=jnp.float32) + ba_ref[...]
    e = jnp.exp(logits - jnp.max(logits, axis=-1, keepdims=True))
    attn = e / jnp.dot(e, sblk_ref[...], preferred_element_type=jnp.float32, precision=jax.lax.Precision.HIGHEST)

    offx = jnp.tanh(jnp.dot(u_r, wdx_ref[...], preferred_element_type=jnp.float32)
                    + bdx_ref[...]) * sig_ref[...]
    offy = jnp.tanh(jnp.dot(u_r, wdy_ref[...], preferred_element_type=jnp.float32)
                    + bdy_ref[...]) * sig_ref[...]
    x = apx * invstr_ref[...] + offx
    y = apy * invstr_ref[...] + offy
    wl = wl_ref[...]
    x0 = jnp.floor(x)
    y0 = jnp.floor(y)
    fx = x - x0
    fy = y - y0
    vx0 = ((x0 >= 0.0) & (x0 <= wl - 1.0)).astype(jnp.float32)
    vx1 = ((x0 + 1.0 >= 0.0) & (x0 + 1.0 <= wl - 1.0)).astype(jnp.float32)
    vy0 = ((y0 >= 0.0) & (y0 <= wl - 1.0)).astype(jnp.float32)
    vy1 = ((y0 + 1.0 >= 0.0) & (y0 + 1.0 <= wl - 1.0)).astype(jnp.float32)
    xc0 = jnp.clip(x0, 0.0, wl - 1.0)
    xc1 = jnp.clip(x0 + 1.0, 0.0, wl - 1.0)
    yc0 = jnp.clip(y0, 0.0, wl - 1.0)
    yc1 = jnp.clip(y0 + 1.0, 0.0, wl - 1.0)
    base = loff_ref[...] + b * np.float32(NPOS)
    hd = head_ref[...]
    i0_ref[0] = ((base + yc0 * wl + xc0) * 6.0 + hd).astype(jnp.int32)
    i1_ref[0] = ((base + yc0 * wl + xc1) * 6.0 + hd).astype(jnp.int32)
    i2_ref[0] = ((base + yc1 * wl + xc0) * 6.0 + hd).astype(jnp.int32)
    i3_ref[0] = ((base + yc1 * wl + xc1) * 6.0 + hd).astype(jnp.int32)
    pt = pt_ref[...]
    scatter96 = lambda w: jnp.dot(w, pt, preferred_element_type=jnp.float32, precision=jax.lax.Precision.HIGHEST)
    w0_ref[0] = scatter96(attn * (1.0 - fx) * (1.0 - fy) * vx0 * vy0)
    w1_ref[0] = scatter96(attn * fx * (1.0 - fy) * vx1 * vy0)
    w2_ref[0] = scatter96(attn * (1.0 - fx) * fy * vx0 * vy1)
    w3_ref[0] = scatter96(attn * fx * fy * vx1 * vy1)


def _run_meta(hx, idxf, qcf, g, wu, bu, lng, lnb,
              wdx, wdy, bdx, bdy, wa, ba, sblk,
              sig, invstr, wl, loff, hd, fxr, fyr, shr, ptm, interpret=False):
    B = g.shape[0]
    n_t = hx.shape[1] // TT
    rep = lambda shp: pl.BlockSpec(shp, lambda b, i: (0, 0))
    out_sp = pl.BlockSpec((1, TT, J), lambda b, i: (b, i, 0))
    out_sp96 = pl.BlockSpec((1, TT, 96), lambda b, i: (b, i, 0))
    out_st = jax.ShapeDtypeStruct((B, n_t * TT, J), jnp.int32)
    out_sf = jax.ShapeDtypeStruct((B, n_t * TT, 96), jnp.float32)
    return pl.pallas_call(
        _meta_kernel,
        grid=(B, n_t),
        in_specs=[
            pl.BlockSpec((1, TT, D_MODEL), lambda b, i: (b, i, 0)),
            pl.BlockSpec((1, TT, 1), lambda b, i: (b, i, 0)),
            pl.BlockSpec((1, TT, 2), lambda b, i: (b, i, 0)),
            pl.BlockSpec((1, 256, D_MODEL), lambda b, i: (b, 0, 0)),
            rep((416, D_MODEL)),
            rep((1, D_MODEL)), rep((1, D_MODEL)), rep((1, D_MODEL)),
            rep((D_MODEL, J)), rep((D_MODEL, J)), rep((1, J)), rep((1, J)),
            rep((D_MODEL, J)), rep((1, J)), rep((J, J)),
            rep((1, J)), rep((1, J)), rep((1, J)), rep((1, J)), rep((1, J)),
            rep((1, 32)), rep((1, 32)), rep((1, 32)), rep((J, 96)),
        ],
        out_specs=[out_sp] * 4 + [out_sp96] * 4,
        out_shape=[out_st] * 4 + [out_sf] * 4,
        interpret=interpret,
    )(hx, idxf, qcf, g, wu, bu, lng, lnb,
      wdx, wdy, bdx, bdy, wa, ba, sblk,
      sig, invstr, wl, loff, hd, fxr, fyr, shr, ptm)


def _sc_gather(table, i0, i1, i2, i3, w0, w1, w2, w3):
    mesh = plsc.VectorSubcoreMesh(core_axis_name="c", subcore_axis_name="s")

    @functools.partial(
        pl.kernel, mesh=mesh,
        compiler_params=pltpu.CompilerParams(use_tc_tiling_on_sc=False, needs_layout_passes=False),
        out_type=jax.ShapeDtypeStruct((T_TOTAL, D_MODEL), jnp.float32),
        scratch_types=(
            [pltpu.VMEM((CH, J), jnp.int32) for _ in range(4)]
            + [pltpu.VMEM((CH, N_HEAD, 16), jnp.float32) for _ in range(4)]
            + [pltpu.VMEM((J, D_HEAD), jnp.bfloat16) for _ in range(8)]
            + [pltpu.VMEM((CH, D_MODEL), jnp.float32),
               pltpu.SemaphoreType.DMA, pltpu.SemaphoreType.DMA,
               pltpu.SemaphoreType.DMA]
        ),
    )
    def k(tab_hbm, i0h, i1h, i2h, i3h, w0h, w1h, w2h, w3h, out_hbm,
          ib0, ib1, ib2, ib3, wb0, wb1, wb2, wb3,
          ra0, ra1, ra2, ra3, rb0, rb1, rb2, rb3,
          ob, msem, gsa, gsb):
        wid = lax.axis_index("s") * 2 + lax.axis_index("c")
        tbase = wid * TPW
        ibufs = (ib0, ib1, ib2, ib3)
        wbufs = (wb0, wb1, wb2, wb3)
        rowsA = (ra0, ra1, ra2, ra3)
        rowsB = (rb0, rb1, rb2, rb3)

        def fire(i, rows):
            for t in range(4):
                pltpu.async_copy(tab_hbm.at[ibufs[t].at[i]], rows[t], gsa if rows is rowsA else gsb)

        def drain(rows):
            sem = gsa if rows is rowsA else gsb
            for t in range(4):
                pltpu.make_async_copy(tab_hbm.at[ibufs[t].at[0]], rows[t], sem).wait()

        def compute(i, rows):
            for h in range(N_HEAD):
                los = []
                his = []
                for t in range(4):
                    wv = wbufs[t][i, h]
                    lo = jnp.zeros((16,), jnp.float32)
                    hi = jnp.zeros((16,), jnp.float32)
                    for lm in range(12):
                        j = h * 12 + lm
                        w = wv[lm]
                        vi = plsc.bitcast(rows[t][j], jnp.int32)
                        lo = lo + w * plsc.bitcast(vi << 16, jnp.float32)
                        hi = hi + w * plsc.bitcast(vi, jnp.float32)
                    los.append(lo)
                    his.append(hi)
                ob[i, h * D_HEAD:h * D_HEAD + 16] = (los[0] + los[1]) + (los[2] + los[3])
                ob[i, h * D_HEAD + 16:h * D_HEAD + 32] = (his[0] + his[1]) + (his[2] + his[3])

        def chunk_body(c, _):
            base = tbase + c * CH
            srcs = (i0h, i1h, i2h, i3h, w0h, w1h, w2h, w3h)
            dsts = (ib0, ib1, ib2, ib3, wb0, wb1, wb2, wb3)
            for s, d in zip(srcs, dsts):
                pltpu.async_copy(s.at[pl.ds(base, CH)], d, msem)
            for s, d in zip(srcs, dsts):
                pltpu.make_async_copy(s.at[pl.ds(base, CH)], d, msem).wait()
            fire(0, rowsA)

            def pair_body(p, _):
                iA = 2 * p
                iB = 2 * p + 1
                fire(iB, rowsB)
                drain(rowsA)
                compute(iA, rowsA)

                @pl.when(p < CH // 2 - 1)
                def _():
                    fire(iB + 1, rowsA)
                drain(rowsB)
                compute(iB, rowsB)
                return 0

            lax.fori_loop(0, CH // 2, pair_body, 0)
            pltpu.async_copy(ob, out_hbm.at[pl.ds(base, CH)], msem)
            pltpu.make_async_copy(ob, out_hbm.at[pl.ds(base, CH)], msem).wait()
            return 0

        lax.fori_loop(0, TPW // CH, chunk_body, 0)

    return k(table, i0, i1, i2, i3, w0, w1, w2, w3)


def _proj_kernel(hr_ref, wo_ref, bo_ref, out_ref):
    out_ref[...] = jnp.dot(hr_ref[...], wo_ref[...],
                           preferred_element_type=jnp.float32) + bo_ref[...]


def _run_proj(hr, w_o, bias, interpret=False):
    return pl.pallas_call(
        _proj_kernel,
        grid=(T_TOTAL // 2048,),
        in_specs=[
            pl.BlockSpec((2048, D_MODEL), lambda i: (i, 0)),
            pl.BlockSpec((D_MODEL, D_MODEL), lambda i: (0, 0)),
            pl.BlockSpec((1, D_MODEL), lambda i: (0, 0)),
        ],
        out_specs=pl.BlockSpec((2048, D_MODEL), lambda i: (i, 0)),
        out_shape=jax.ShapeDtypeStruct((T_TOTAL, D_MODEL), jnp.float32),
        interpret=interpret,
    )(hr, w_o, bias)


def _prep(h, top_indices, query_coords, g, L2_proj, L3_proj, L4_proj,
          w_u, b_u, w_delta, b_delta):
    B, K, d = h.shape
    R = top_indices.shape[2]
    hx = jnp.broadcast_to(h[:, :, None, :], (B, K, R, d)).reshape(B, K * R, d)
    idxf = top_indices.reshape(B, K * R, 1)
    qcf = jnp.broadcast_to(query_coords[:, :, None, :], (B, K, R, 2)).reshape(B, K * R, 2)
    tabs = []
    for F in (L2_proj, L3_proj, L4_proj):
        Hf, Wf = F.shape[2], F.shape[3]
        tabs.append(jnp.transpose(F, (0, 2, 3, 1)).reshape(B, Hf * Wf, d))
    table = jnp.concatenate(tabs, axis=1).reshape(B * NPOS * N_HEAD, D_HEAD)
    perm = np.empty(D_HEAD, np.int64)
    perm[0::2] = np.arange(16)
    perm[1::2] = np.arange(16) + 16
    table = table[:, perm].astype(jnp.bfloat16)
    wdx = w_delta[:, 0::2]
    wdy = w_delta[:, 1::2]
    bdx = b_delta[0::2].reshape(1, J)
    bdy = b_delta[1::2].reshape(1, J)
    bu = b_u.reshape(1, d)
    return hx, idxf, qcf, table, bu, wdx, wdy, bdx, bdy


def _consts():
    j = np.arange(J)
    l = (j % (N_LEVELS * N_POINTS)) // N_POINTS
    hh = j // (N_LEVELS * N_POINTS)
    sig = np.array(SIGMAS, np.float32)[l][None]
    invstr = (1.0 / np.array(STRIDES, np.float32))[l][None]
    wl = np.array(LVL_W, np.float32)[l][None]
    loff = np.array(LVL_OFF, np.float32)[l][None]
    hd = hh[None].astype(np.float32)
    sb = np.kron(np.eye(N_HEAD, dtype=np.float32), np.ones((12, 12), np.float32))
    fr = 2.0 ** np.arange(8)
    fxr = 2.0 * np.pi * np.concatenate([fr, fr, np.zeros(16)])[None].astype(np.float32)
    fyr = 2.0 * np.pi * np.concatenate([np.zeros(16), fr, fr])[None].astype(np.float32)
    shr = np.concatenate([np.zeros(8), np.ones(8),
                          np.zeros(8), np.ones(8)])[None].astype(np.float32)
    pt = np.zeros((J, 96), np.float32)
    for jj in range(J):
        hq, lm = jj // 12, jj % 12
        pt[jj, hq * 16 + lm] = 1.0
    return (jnp.asarray(sig), jnp.asarray(invstr), jnp.asarray(wl),
            jnp.asarray(loff), jnp.asarray(hd), jnp.asarray(sb),
            jnp.asarray(fxr), jnp.asarray(fyr), jnp.asarray(shr),
            jnp.asarray(pt))


def kernel(h, top_indices, query_coords, g, L2_proj, L3_proj, L4_proj,
           w_u, b_u, ln_g, ln_b, w_delta, b_delta, w_a, b_a, w_o, b_o, e_deform):
    B, K, d = h.shape
    R = top_indices.shape[2]
    (hx, idxf, qcf, table, bu,
     wdx, wdy, bdx, bdy) = _prep(h, top_indices, query_coords, g,
                                 L2_proj, L3_proj, L4_proj, w_u, b_u,
                                 w_delta, b_delta)
    sig, invstr, wl, loff, hd, sb, fxr, fyr, shr, ptm = _consts()
    i0, i1, i2, i3, w0, w1, w2, w3 = _run_meta(
        hx, idxf, qcf, g, w_u, bu,
        ln_g.reshape(1, d), ln_b.reshape(1, d),
        wdx, wdy, bdx, bdy, w_a, b_a.reshape(1, J), sb,
        sig, invstr, wl, loff, hd, fxr, fyr, shr, ptm)
    rs = lambda a: a.reshape(T_TOTAL, J)
    rw = lambda a: a.reshape(T_TOTAL, N_HEAD, 16)
    hr = _sc_gather(table, rs(i0), rs(i1), rs(i2), rs(i3),
                    rw(w0), rw(w1), rw(w2), rw(w3))
    bias = (b_o + e_deform.reshape(d)).reshape(1, d)
    out = _run_proj(hr, w_o, bias)
    return out.reshape(B, K, R, d)
